# baseline (speedup 1.0000x reference)
import functools

import jax
import jax.numpy as jnp
from jax import lax
from jax.experimental import pallas as pl
from jax.experimental.pallas import tpu as pltpu
from jax.experimental.pallas import tpu_sc as plsc

N = 10000
E = 320000
D = 128
NB = 300
HID = 64
G = 64
GAMMA = 10.0
DMIN = 0.0
DMAX = 30.0

BE = 1280
BN = 1000
NW = 32
EPW = E // NW
CH = 80
NCH = EPW // CH
NPAD = 10240
RPS = NPAD // 16
ZR = 128

_LOG2 = 0.6931471805599453


def _ssp(v):
    return jnp.maximum(v, 0.0) + jnp.log1p(jnp.exp(-jnp.abs(v))) - _LOG2


def _filter_body(dist_ref, w1_ref, b1_ref, w2_ref, b2_ref, out_ref):
    d = dist_ref[0, 0, :]
    step = (DMAX - DMIN) / (NB - 1)
    col = lax.broadcasted_iota(jnp.int32, (BE, NB), 1).astype(jnp.float32)
    diff = d[:, None] - (DMIN + step * col)
    rbf = jnp.exp((-GAMMA) * diff * diff)
    h1 = lax.dot_general(rbf, w1_ref[...], (((1,), (1,)), ((), ())),
                         preferred_element_type=jnp.float32)
    h1 = _ssp(h1 + b1_ref[0, :][None, :])
    h2 = lax.dot_general(h1, w2_ref[...], (((1,), (1,)), ((), ())),
                         preferred_element_type=jnp.float32)
    out_ref[...] = _ssp(h2 + b2_ref[0, :][None, :]).astype(jnp.bfloat16)


_filter = pl.pallas_call(
    _filter_body,
    grid=(E // BE,),
    in_specs=[
        pl.BlockSpec((1, 1, BE), lambda i: (i, 0, 0)),
        pl.BlockSpec((HID, NB), lambda i: (0, 0)),
        pl.BlockSpec((1, HID), lambda i: (0, 0)),
        pl.BlockSpec((D, HID), lambda i: (0, 0)),
        pl.BlockSpec((1, D), lambda i: (0, 0)),
    ],
    out_specs=pl.BlockSpec((BE, D), lambda i: (i, 0)),
    out_shape=jax.ShapeDtypeStruct((E, D), jnp.bfloat16),
)

_WPERM = [32 * g + 16 * p + j for g in range(4) for j in range(16)
          for p in range(2)]


def _mm_body(h_ref, w_ref, b_ref, out_ref):
    y = lax.dot_general(h_ref[...], w_ref[...], (((1,), (1,)), ((), ())),
                        preferred_element_type=jnp.float32)
    out_ref[...] = y + b_ref[0, :][None, :]


_mm = pl.pallas_call(
    _mm_body,
    grid=(N // BN,),
    in_specs=[
        pl.BlockSpec((BN, D), lambda i: (i, 0)),
        pl.BlockSpec((D, D), lambda i: (0, 0)),
        pl.BlockSpec((1, D), lambda i: (0, 0)),
    ],
    out_specs=pl.BlockSpec((BN, D), lambda i: (i, 0)),
    out_shape=jax.ShapeDtypeStruct((N, D), jnp.float32),
)


def _update_body(h_ref, agg_ref, w_ref, b_ref, out_ref):
    a = agg_ref[0] + agg_ref[1]
    y = lax.dot_general(a, w_ref[...], (((1,), (1,)), ((), ())),
                        preferred_element_type=jnp.float32)
    out_ref[...] = h_ref[...] + _ssp(y + b_ref[0, :][None, :])


_update = pl.pallas_call(
    _update_body,
    grid=(N // BN,),
    in_specs=[
        pl.BlockSpec((BN, D), lambda i: (i, 0)),
        pl.BlockSpec((2, BN, D), lambda i: (0, i, 0)),
        pl.BlockSpec((D, D), lambda i: (0, 0)),
        pl.BlockSpec((1, D), lambda i: (0, 0)),
    ],
    out_specs=pl.BlockSpec((BN, D), lambda i: (i, 0)),
    out_shape=jax.ShapeDtypeStruct((N, D), jnp.float32),
)


def _head_body(h_ref, batch_ref, w1_ref, b1_ref, w2_ref, b2_ref, out_ref):
    i = pl.program_id(0)
    t = lax.dot_general(h_ref[...], w1_ref[...], (((1,), (1,)), ((), ())),
                        preferred_element_type=jnp.float32)
    t = _ssp(t + b1_ref[0, :][None, :])
    y = lax.dot_general(t, w2_ref[...], (((1,), (1,)), ((), ())),
                        preferred_element_type=jnp.float32)
    y = y + b2_ref[0, :][None, :]
    bidx = batch_ref[0, 0, :]
    onehot = (bidx[:, None]
              == lax.broadcasted_iota(jnp.int32, (BN, G), 1)).astype(jnp.float32)
    contrib = lax.dot_general(onehot, y, (((0,), (0,)), ((), ())),
                              preferred_element_type=jnp.float32)

    @pl.when(i == 0)
    def _():
        out_ref[...] = contrib

    @pl.when(i > 0)
    def _():
        out_ref[...] = out_ref[...] + contrib


_head = pl.pallas_call(
    _head_body,
    grid=(N // BN,),
    in_specs=[
        pl.BlockSpec((BN, D), lambda i: (i, 0)),
        pl.BlockSpec((1, 1, BN), lambda i: (i, 0, 0)),
        pl.BlockSpec((D, D), lambda i: (0, 0)),
        pl.BlockSpec((1, D), lambda i: (0, 0)),
        pl.BlockSpec((D, D), lambda i: (0, 0)),
        pl.BlockSpec((1, D), lambda i: (0, 0)),
    ],
    out_specs=pl.BlockSpec((G, D), lambda i: (0, 0)),
    out_shape=jax.ShapeDtypeStruct((G, D), jnp.float32),
)


@functools.cache
def _make_sc_gms():
    mesh = plsc.VectorSubcoreMesh(core_axis_name="c", subcore_axis_name="s")
    return functools.partial(
        pl.kernel,
        out_type=jax.ShapeDtypeStruct((2, NPAD, D), jnp.float32),
        mesh=mesh,
        compiler_params=pltpu.CompilerParams(needs_layout_passes=False),
        scratch_types=[
            pltpu.VMEM((CH,), jnp.int32),
            pltpu.VMEM((CH,), jnp.int32),
            pltpu.VMEM((CH,), jnp.int32),
            pltpu.VMEM((CH,), jnp.int32),
            pltpu.VMEM((CH, D), jnp.float32),
            pltpu.VMEM((CH, D), jnp.float32),
            pltpu.VMEM((CH * D // 2,), jnp.int32),
            pltpu.VMEM((CH * D // 2,), jnp.int32),
            pltpu.VMEM_SHARED((NPAD, D), jnp.float32),
            pltpu.SemaphoreType.DMA,
            pltpu.SemaphoreType.DMA,
            pltpu.SemaphoreType.DMA,
            pltpu.SemaphoreType.DMA,
            pltpu.SemaphoreType.DMA,
            pltpu.SemaphoreType.DMA,
        ],
    )(_sc_gms_body)


def _sc_gms_body(m_hbm, w_hbm, src_hbm, dst_hbm, out_hbm,
                 sidx0, sidx1, didx0, didx1, rows0, rows1, wrows0, wrows1,
                 agg, isem0, isem1, gsem0, gsem1, wsem0, wsem1):
    c = lax.axis_index("c")
    s = lax.axis_index("s")
    wid = s * 2 + c

    sidx = (sidx0, sidx1)
    didx = (didx0, didx1)
    rows = (rows0, rows1)
    wrows = (wrows0, wrows1)
    isem = (isem0, isem1)
    gsem = (gsem0, gsem1)
    wsem = (wsem0, wsem1)

    zero16 = jnp.zeros((16,), jnp.float32)

    def zrow(r, _):
        for g in range(D // 16):
            rows0[r, pl.ds(g * 16, 16)] = zero16
        return 0

    lax.fori_loop(0, CH, zrow, 0)

    def zcopy(j, _):
        pltpu.sync_copy(rows0, agg.at[pl.ds(s * RPS + j * CH, CH)])
        return 0

    lax.fori_loop(0, RPS // CH, zcopy, 0)
    plsc.subcore_barrier()

    ebase = wid * EPW

    def idx_fetch(b, i):
        pltpu.async_copy(src_hbm.at[wid, i], sidx[b], isem[b])
        pltpu.async_copy(dst_hbm.at[wid, i], didx[b], isem[b])

    def idx_wait(b, i):
        pltpu.make_async_copy(src_hbm.at[wid, i], sidx[b], isem[b]).wait()
        pltpu.make_async_copy(dst_hbm.at[wid, i], didx[b], isem[b]).wait()

    def data_fetch(b, i):
        pltpu.async_copy(m_hbm.at[sidx[b]], rows[b], gsem[b])
        pltpu.async_copy(
            w_hbm.at[pl.ds((ebase + i * CH) * (D // 2), CH * D // 2)],
            wrows[b], wsem[b])

    def flush(b, i):
        pltpu.make_async_copy(m_hbm.at[sidx[b]], rows[b], gsem[b]).wait()
        pltpu.make_async_copy(
            w_hbm.at[pl.ds((ebase + i * CH) * (D // 2), CH * D // 2)],
            wrows[b], wsem[b]).wait()

        def mrow(r, _):
            for g in range(D // 32):
                wv = wrows[b][pl.ds(r * (D // 2) + 16 * g, 16)]
                lo = plsc.bitcast(lax.shift_left(wv, 16), jnp.float32)
                hi = plsc.bitcast(lax.bitwise_and(wv, jnp.int32(-65536)),
                                  jnp.float32)
                sl0 = pl.ds(32 * g, 16)
                sl1 = pl.ds(32 * g + 16, 16)
                rows[b][r, sl0] = rows[b][r, sl0] * lo
                rows[b][r, sl1] = rows[b][r, sl1] * hi
            return 0

        lax.fori_loop(0, CH, mrow, 0)
        pltpu.sync_copy(rows[b], agg.at[didx[b]], add=True)

    idx_fetch(0, 0)
    idx_wait(0, 0)
    data_fetch(0, 0)
    idx_fetch(1, 1)

    def pair(k, _):
        i = 2 * k
        idx_wait(1, i + 1)
        data_fetch(1, i + 1)
        flush(0, i)
        idx_fetch(0, i + 2)
        flush(1, i + 1)

        @pl.when(i + 3 < NCH)
        def _():
            idx_fetch(1, i + 3)

        idx_wait(0, i + 2)
        data_fetch(0, i + 2)
        return 0

    lax.fori_loop(0, (NCH - 1) // 2, pair, 0)
    flush(0, NCH - 1)
    plsc.subcore_barrier()

    def ocopy(j, _):
        r0 = s * RPS + j * CH
        pltpu.sync_copy(agg.at[pl.ds(r0, CH)], rows0)
        pltpu.sync_copy(rows0, out_hbm.at[c, pl.ds(r0, CH)])
        return 0

    lax.fori_loop(0, RPS // CH, ocopy, 0)


def kernel(x, edge_index, dist, batch, params):
    src = edge_index[0].reshape(NW, NCH, CH)
    dst = edge_index[1].reshape(NW, NCH, CH)
    dist3 = dist.reshape(E // BE, 1, BE)
    batch3 = batch.reshape(N // BN, 1, BN)

    def b2d(b):
        return b.reshape(1, -1)

    wperm = jnp.array(_WPERM, dtype=jnp.int32)
    ws = []
    for p in params["convs"]:
        wb = _filter(dist3, p["f_W1"], b2d(p["f_b1"]),
                     p["f_W2"][wperm, :], b2d(p["f_b2"][wperm]))
        ws.append(lax.bitcast_convert_type(
            wb.reshape(E, D // 2, 2), jnp.int32).reshape(-1))

    h = _mm(x, params["emb"]["W"], b2d(params["emb"]["b"]))
    for p, w in zip(params["convs"], ws):
        m = _mm(h, p["lin1_W"], b2d(jnp.zeros((D,), jnp.float32)))
        agg2 = _make_sc_gms()(m, w, src, dst)
        h = _update(h, agg2, p["lin2_W"], b2d(p["lin2_b"]))

    return _head(h, batch3, params["head"]["W1"], b2d(params["head"]["b1"]),
                 params["head"]["W2"], b2d(params["head"]["b2"]))


# speedup vs baseline: 3.3385x; 3.3385x over previous
import functools

import jax
import jax.numpy as jnp
from jax import lax
from jax.experimental import pallas as pl
from jax.experimental.pallas import tpu as pltpu
from jax.experimental.pallas import tpu_sc as plsc

N = 10000
E = 320000
D = 128
NB = 300
HID = 64
G = 64
GAMMA = 10.0
DMIN = 0.0
DMAX = 30.0

BE = 1280
BN = 1000
NW = 32
EPW = E // NW
CH = 80
NCH = EPW // CH
NPAD = 10240
RPS = NPAD // 16
ZR = 128

_LOG2 = 0.6931471805599453


def _ssp(v):
    return jnp.maximum(v, 0.0) + jnp.log1p(jnp.exp(-jnp.abs(v))) - _LOG2


def _filter_body(dist_ref, w1_ref, b1_ref, w2_ref, b2_ref, out_ref):
    d = dist_ref[0, 0, :]
    step = (DMAX - DMIN) / (NB - 1)
    col = lax.broadcasted_iota(jnp.int32, (BE, NB), 1).astype(jnp.float32)
    diff = d[:, None] - (DMIN + step * col)
    rbf = jnp.exp((-GAMMA) * diff * diff)
    h1 = lax.dot_general(rbf, w1_ref[...], (((1,), (1,)), ((), ())),
                         preferred_element_type=jnp.float32)
    h1 = _ssp(h1 + b1_ref[0, :][None, :])
    h2 = lax.dot_general(h1, w2_ref[...], (((1,), (1,)), ((), ())),
                         preferred_element_type=jnp.float32)
    out_ref[...] = _ssp(h2 + b2_ref[0, :][None, :])


_filter = pl.pallas_call(
    _filter_body,
    grid=(E // BE,),
    in_specs=[
        pl.BlockSpec((1, 1, BE), lambda i: (i, 0, 0)),
        pl.BlockSpec((HID, NB), lambda i: (0, 0)),
        pl.BlockSpec((1, HID), lambda i: (0, 0)),
        pl.BlockSpec((D, HID), lambda i: (0, 0)),
        pl.BlockSpec((1, D), lambda i: (0, 0)),
    ],
    out_specs=pl.BlockSpec((BE, D), lambda i: (i, 0)),
    out_shape=jax.ShapeDtypeStruct((E, D), jnp.float32),
)


def _mm_body(h_ref, w_ref, b_ref, out_ref):
    y = lax.dot_general(h_ref[...], w_ref[...], (((1,), (1,)), ((), ())),
                        preferred_element_type=jnp.float32)
    out_ref[...] = y + b_ref[0, :][None, :]


_mm = pl.pallas_call(
    _mm_body,
    grid=(N // BN,),
    in_specs=[
        pl.BlockSpec((BN, D), lambda i: (i, 0)),
        pl.BlockSpec((D, D), lambda i: (0, 0)),
        pl.BlockSpec((1, D), lambda i: (0, 0)),
    ],
    out_specs=pl.BlockSpec((BN, D), lambda i: (i, 0)),
    out_shape=jax.ShapeDtypeStruct((N, D), jnp.float32),
)


def _update_body(h_ref, agg_ref, w_ref, b_ref, out_ref):
    a = agg_ref[0] + agg_ref[1]
    y = lax.dot_general(a, w_ref[...], (((1,), (1,)), ((), ())),
                        preferred_element_type=jnp.float32)
    out_ref[...] = h_ref[...] + _ssp(y + b_ref[0, :][None, :])


_update = pl.pallas_call(
    _update_body,
    grid=(N // BN,),
    in_specs=[
        pl.BlockSpec((BN, D), lambda i: (i, 0)),
        pl.BlockSpec((2, BN, D), lambda i: (0, i, 0)),
        pl.BlockSpec((D, D), lambda i: (0, 0)),
        pl.BlockSpec((1, D), lambda i: (0, 0)),
    ],
    out_specs=pl.BlockSpec((BN, D), lambda i: (i, 0)),
    out_shape=jax.ShapeDtypeStruct((N, D), jnp.float32),
)


def _head_body(h_ref, batch_ref, w1_ref, b1_ref, w2_ref, b2_ref, out_ref):
    i = pl.program_id(0)
    t = lax.dot_general(h_ref[...], w1_ref[...], (((1,), (1,)), ((), ())),
                        preferred_element_type=jnp.float32)
    t = _ssp(t + b1_ref[0, :][None, :])
    y = lax.dot_general(t, w2_ref[...], (((1,), (1,)), ((), ())),
                        preferred_element_type=jnp.float32)
    y = y + b2_ref[0, :][None, :]
    bidx = batch_ref[0, 0, :]
    onehot = (bidx[:, None]
              == lax.broadcasted_iota(jnp.int32, (BN, G), 1)).astype(jnp.float32)
    contrib = lax.dot_general(onehot, y, (((0,), (0,)), ((), ())),
                              preferred_element_type=jnp.float32)

    @pl.when(i == 0)
    def _():
        out_ref[...] = contrib

    @pl.when(i > 0)
    def _():
        out_ref[...] = out_ref[...] + contrib


_head = pl.pallas_call(
    _head_body,
    grid=(N // BN,),
    in_specs=[
        pl.BlockSpec((BN, D), lambda i: (i, 0)),
        pl.BlockSpec((1, 1, BN), lambda i: (i, 0, 0)),
        pl.BlockSpec((D, D), lambda i: (0, 0)),
        pl.BlockSpec((1, D), lambda i: (0, 0)),
        pl.BlockSpec((D, D), lambda i: (0, 0)),
        pl.BlockSpec((1, D), lambda i: (0, 0)),
    ],
    out_specs=pl.BlockSpec((G, D), lambda i: (0, 0)),
    out_shape=jax.ShapeDtypeStruct((G, D), jnp.float32),
)


@functools.cache
def _make_sc_gms():
    mesh = plsc.VectorSubcoreMesh(core_axis_name="c", subcore_axis_name="s")
    return functools.partial(
        pl.kernel,
        out_type=jax.ShapeDtypeStruct((2, NPAD, D), jnp.float32),
        mesh=mesh,
        scratch_types=[
            pltpu.VMEM((CH,), jnp.int32),
            pltpu.VMEM((CH,), jnp.int32),
            pltpu.VMEM((CH,), jnp.int32),
            pltpu.VMEM((CH,), jnp.int32),
            pltpu.VMEM((CH, D), jnp.float32),
            pltpu.VMEM((CH, D), jnp.float32),
            pltpu.VMEM((CH, D), jnp.float32),
            pltpu.VMEM((CH, D), jnp.float32),
            pltpu.VMEM_SHARED((NPAD, D), jnp.float32),
            pltpu.SemaphoreType.DMA,
            pltpu.SemaphoreType.DMA,
            pltpu.SemaphoreType.DMA,
            pltpu.SemaphoreType.DMA,
            pltpu.SemaphoreType.DMA,
            pltpu.SemaphoreType.DMA,
        ],
    )(_sc_gms_body)


def _sc_gms_body(m_hbm, w_hbm, src_hbm, dst_hbm, out_hbm,
                 sidx0, sidx1, didx0, didx1, rows0, rows1, wrows0, wrows1,
                 agg, isem0, isem1, gsem0, gsem1, wsem0, wsem1):
    c = lax.axis_index("c")
    s = lax.axis_index("s")
    wid = s * 2 + c

    sidx = (sidx0, sidx1)
    didx = (didx0, didx1)
    rows = (rows0, rows1)
    wrows = (wrows0, wrows1)
    isem = (isem0, isem1)
    gsem = (gsem0, gsem1)
    wsem = (wsem0, wsem1)

    zero16 = jnp.zeros((16,), jnp.float32)

    def zrow(r, _):
        for g in range(D // 16):
            rows0[r, pl.ds(g * 16, 16)] = zero16
        return 0

    lax.fori_loop(0, CH, zrow, 0)

    def zcopy(j, _):
        pltpu.sync_copy(rows0, agg.at[pl.ds(s * RPS + j * CH, CH)])
        return 0

    lax.fori_loop(0, RPS // CH, zcopy, 0)
    plsc.subcore_barrier()

    ebase = wid * EPW

    def idx_fetch(b, i):
        pltpu.async_copy(src_hbm.at[wid, i], sidx[b], isem[b])
        pltpu.async_copy(dst_hbm.at[wid, i], didx[b], isem[b])

    def idx_wait(b, i):
        pltpu.make_async_copy(src_hbm.at[wid, i], sidx[b], isem[b]).wait()
        pltpu.make_async_copy(dst_hbm.at[wid, i], didx[b], isem[b]).wait()

    def data_fetch(b, i):
        pltpu.async_copy(m_hbm.at[sidx[b]], rows[b], gsem[b])
        pltpu.async_copy(w_hbm.at[pl.ds(ebase + i * CH, CH)], wrows[b],
                         wsem[b])

    def flush(b, i):
        pltpu.make_async_copy(m_hbm.at[sidx[b]], rows[b], gsem[b]).wait()
        pltpu.make_async_copy(w_hbm.at[pl.ds(ebase + i * CH, CH)], wrows[b],
                              wsem[b]).wait()

        def mrow(r, _):
            for g in range(D // 16):
                sl = pl.ds(g * 16, 16)
                rows[b][r, sl] = rows[b][r, sl] * wrows[b][r, sl]
            return 0

        lax.fori_loop(0, CH, mrow, 0)
        pltpu.sync_copy(rows[b], agg.at[didx[b]], add=True)

    idx_fetch(0, 0)
    idx_wait(0, 0)
    data_fetch(0, 0)
    idx_fetch(1, 1)

    def pair(k, _):
        i = 2 * k
        idx_wait(1, i + 1)
        data_fetch(1, i + 1)
        flush(0, i)
        idx_fetch(0, i + 2)
        flush(1, i + 1)

        @pl.when(i + 3 < NCH)
        def _():
            idx_fetch(1, i + 3)

        idx_wait(0, i + 2)
        data_fetch(0, i + 2)
        return 0

    lax.fori_loop(0, (NCH - 1) // 2, pair, 0)
    flush(0, NCH - 1)
    plsc.subcore_barrier()

    def ocopy(j, _):
        r0 = s * RPS + j * CH
        pltpu.sync_copy(agg.at[pl.ds(r0, CH)], rows0)
        pltpu.sync_copy(rows0, out_hbm.at[c, pl.ds(r0, CH)])
        return 0

    lax.fori_loop(0, RPS // CH, ocopy, 0)


def kernel(x, edge_index, dist, batch, params):
    src = edge_index[0].reshape(NW, NCH, CH)
    dst = edge_index[1].reshape(NW, NCH, CH)
    dist3 = dist.reshape(E // BE, 1, BE)
    batch3 = batch.reshape(N // BN, 1, BN)

    def b2d(b):
        return b.reshape(1, -1)

    ws = [
        _filter(dist3, p["f_W1"], b2d(p["f_b1"]), p["f_W2"], b2d(p["f_b2"]))
        for p in params["convs"]
    ]

    h = _mm(x, params["emb"]["W"], b2d(params["emb"]["b"]))
    for p, w in zip(params["convs"], ws):
        m = _mm(h, p["lin1_W"], b2d(jnp.zeros((D,), jnp.float32)))
        agg2 = _make_sc_gms()(m, w, src, dst)
        h = _update(h, agg2, p["lin2_W"], b2d(p["lin2_b"]))

    return _head(h, batch3, params["head"]["W1"], b2d(params["head"]["b1"]),
                 params["head"]["W2"], b2d(params["head"]["b2"]))
